# Initial kernel scaffold; baseline (speedup 1.0000x reference)
import jax
import jax.numpy as jnp
from jax.experimental import pallas as pl


def kernel(x, edge_index, W_in, b_in, W_l, b_l, W_r, W_cls):
    raise NotImplementedError("write your pallas kernel here")


# speedup vs baseline: 5.4386x; 5.4386x over previous
import functools

import jax
import jax.numpy as jnp
from jax import lax
from jax.experimental import pallas as pl
from jax.experimental.pallas import tpu as pltpu
from jax.experimental.pallas import tpu_sc as plsc

_N = 10000
_E = 320000
_D = 128
_H = 128
_C = 20
_TEMP = 10.0

_HE = 144
_NC = 2
_NS = 16
_NW = _NC * _NS
_CHUNK = 128
_NCHUNK = 79
_EPAD = _NW * _NCHUNK * _CHUNK
_RPS = 640
_AROWS = _NS * _RPS

_BN = 2000


def _pre_body(x_ref, win_ref, bin_ref, wr_ref, bl_ref, fe_ref, base_ref):
    xb = x_ref[...]
    feat = jnp.dot(xb, win_ref[...], preferred_element_type=jnp.float32)
    feat = jnp.maximum(feat + bin_ref[...], 0.0)
    col = lax.broadcasted_iota(jnp.int32, (_BN, _HE - _H), 1)
    tail = jnp.where(col == 0, 1.0, 0.0).astype(jnp.float32)
    fe_ref[...] = jnp.concatenate([feat, tail], axis=1)
    base = jnp.dot(feat, wr_ref[...], preferred_element_type=jnp.float32)
    base_ref[...] = base + bl_ref[...]


def _pre(x, w_in, b_in, w_r, b_l):
    grid = _N // _BN
    return pl.pallas_call(
        _pre_body,
        grid=(grid,),
        in_specs=[
            pl.BlockSpec((_BN, _D), lambda i: (i, 0)),
            pl.BlockSpec((_D, _H), lambda i: (0, 0)),
            pl.BlockSpec((1, _H), lambda i: (0, 0)),
            pl.BlockSpec((_H, _H), lambda i: (0, 0)),
            pl.BlockSpec((1, _H), lambda i: (0, 0)),
        ],
        out_specs=[
            pl.BlockSpec((_BN, _HE), lambda i: (i, 0)),
            pl.BlockSpec((_BN, _H), lambda i: (i, 0)),
        ],
        out_shape=[
            jax.ShapeDtypeStruct((_N, _HE), jnp.float32),
            jax.ShapeDtypeStruct((_N, _H), jnp.float32),
        ],
    )(x, w_in, b_in, w_r, b_l)


def _sc_body(feat_hbm, srcs_hbm, dsts_hbm, out_hbm,
             src_v, dst_v, rows_v, acc_sh, sem):
    c = lax.axis_index("c")
    s = lax.axis_index("s")
    w = s * _NC + c

    def zrow(i, _):
        def zcol(j, _):
            rows_v[i, pl.ds(j * 16, 16)] = jnp.zeros((16,), jnp.float32)
            return 0
        return lax.fori_loop(0, _HE // 16, zcol, 0)
    lax.fori_loop(0, _CHUNK, zrow, 0)

    def zcp(t, _):
        pltpu.sync_copy(rows_v,
                        acc_sh.at[pl.ds(s * _RPS + t * _CHUNK, _CHUNK)])
        return 0
    lax.fori_loop(0, _RPS // _CHUNK, zcp, 0)
    plsc.subcore_barrier()

    pltpu.sync_copy(srcs_hbm.at[w], src_v)
    pltpu.sync_copy(dsts_hbm.at[w], dst_v)

    def chunk(j, _):
        pltpu.async_copy(feat_hbm.at[src_v.at[j]], rows_v, sem).wait()
        pltpu.sync_copy(rows_v, acc_sh.at[dst_v.at[j]], add=True)
        return 0
    lax.fori_loop(0, _NCHUNK, chunk, 0)
    plsc.subcore_barrier()

    base = s * _RPS
    pltpu.sync_copy(acc_sh.at[pl.ds(base, _RPS)],
                    out_hbm.at[c, pl.ds(base, _RPS)])


_sc_agg = functools.partial(
    pl.kernel,
    out_type=jax.ShapeDtypeStruct((_NC, _AROWS, _HE), jnp.float32),
    mesh=plsc.VectorSubcoreMesh(core_axis_name="c", subcore_axis_name="s"),
    compiler_params=pltpu.CompilerParams(use_tc_tiling_on_sc=False),
    scratch_types=[
        pltpu.VMEM((_NCHUNK, _CHUNK), jnp.int32),
        pltpu.VMEM((_NCHUNK, _CHUNK), jnp.int32),
        pltpu.VMEM((_CHUNK, _HE), jnp.float32),
        pltpu.VMEM_SHARED((_AROWS, _HE), jnp.float32),
        pltpu.SemaphoreType.DMA,
    ],
)(_sc_body)


def _post_body(p0_ref, p1_ref, base_ref, wl_ref, wcls_ref, out_ref, of_ref):
    acc = p0_ref[:, :_H] + p1_ref[:, :_H]
    cnt = p0_ref[:, _H:_H + 1] + p1_ref[:, _H:_H + 1]
    mean = acc / jnp.maximum(cnt, 1.0)
    of = jnp.dot(mean, wl_ref[...], preferred_element_type=jnp.float32)
    of = of + base_ref[...]
    of_ref[...] = of
    nrm = jnp.sqrt(jnp.sum(of * of, axis=1, keepdims=True))
    xn = of / jnp.maximum(nrm, 1e-12)
    wc = wcls_ref[...]
    wnrm = jnp.sqrt(jnp.sum(wc * wc, axis=0, keepdims=True))
    wn = wc / jnp.maximum(wnrm, 1e-12)
    out_ref[...] = _TEMP * jnp.dot(xn, wn, preferred_element_type=jnp.float32)


def _post(p0, p1, base, w_l, w_cls):
    grid = _N // _BN
    return pl.pallas_call(
        _post_body,
        grid=(grid,),
        in_specs=[
            pl.BlockSpec((_BN, _HE), lambda i: (i, 0)),
            pl.BlockSpec((_BN, _HE), lambda i: (i, 0)),
            pl.BlockSpec((_BN, _H), lambda i: (i, 0)),
            pl.BlockSpec((_H, _H), lambda i: (0, 0)),
            pl.BlockSpec((_H, _C), lambda i: (0, 0)),
        ],
        out_specs=[
            pl.BlockSpec((_BN, _C), lambda i: (i, 0)),
            pl.BlockSpec((_BN, _H), lambda i: (i, 0)),
        ],
        out_shape=[
            jax.ShapeDtypeStruct((_N, _C), jnp.float32),
            jax.ShapeDtypeStruct((_N, _H), jnp.float32),
        ],
    )(p0, p1, base, w_l, w_cls)


def kernel(x, edge_index, W_in, b_in, W_l, b_l, W_r, W_cls):
    feat_ext, base = _pre(x, W_in, b_in.reshape(1, _H),
                          W_r, b_l.reshape(1, _H))

    pad = _EPAD - _E
    src = jnp.concatenate([edge_index[0], jnp.zeros((pad,), jnp.int32)])
    dst = jnp.concatenate([edge_index[1], jnp.full((pad,), _N, jnp.int32)])
    srcs = src.reshape(_NW, _NCHUNK, _CHUNK)
    dsts = dst.reshape(_NW, _NCHUNK, _CHUNK)

    parts = _sc_agg(feat_ext, srcs, dsts)

    out, out_feat = _post(parts[0, :_N], parts[1, :_N], base, W_l, W_cls)
    return (out, out_feat)
